# baseline (device time: 1395624 ns/iter reference)
import jax
import jax.numpy as jnp
from jax import lax
from jax.experimental import pallas as pl
from jax.experimental.pallas import tpu as pltpu

N_DEV = 4


def _ring_hop2(src_r, src_l):

    def body(sr_ref, sl_ref, or_ref, ol_ref, ss_r, rs_r, ss_l, rs_l):
        d = lax.axis_index("i")
        right = lax.rem(d + 1, N_DEV)
        left = lax.rem(d + N_DEV - 1, N_DEV)
        rdma_r = pltpu.make_async_remote_copy(
            src_ref=sr_ref, dst_ref=or_ref, send_sem=ss_r, recv_sem=rs_r,
            device_id=(right,), device_id_type=pl.DeviceIdType.MESH,
        )
        rdma_l = pltpu.make_async_remote_copy(
            src_ref=sl_ref, dst_ref=ol_ref, send_sem=ss_l, recv_sem=rs_l,
            device_id=(left,), device_id_type=pl.DeviceIdType.MESH,
        )
        rdma_r.start()
        rdma_l.start()
        rdma_r.wait()
        rdma_l.wait()

    return pl.pallas_call(
        body,
        out_shape=(
            jax.ShapeDtypeStruct(src_r.shape, src_r.dtype),
            jax.ShapeDtypeStruct(src_l.shape, src_l.dtype),
        ),
        in_specs=[
            pl.BlockSpec(memory_space=pl.ANY),
            pl.BlockSpec(memory_space=pl.ANY),
        ],
        out_specs=(
            pl.BlockSpec(memory_space=pl.ANY),
            pl.BlockSpec(memory_space=pl.ANY),
        ),
        scratch_shapes=[pltpu.SemaphoreType.DMA] * 4,
    )(src_r, src_l)


def _merge_all_gather(scale, recv_a, recv_b, partial, m, n):
    h = m // 2
    tile = 1024

    def body(scale_ref, ra_hbm, rb_hbm, p_hbm, out_ref, sa, ra, sb, rb):
        d = lax.axis_index("i")
        right = lax.rem(d + 1, N_DEV)
        left = lax.rem(d + N_DEV - 1, N_DEV)
        s_val = scale_ref[0]

        def merge_tile(ra_t, rb_t, pa_t, out_t):
            def silu(acc):
                y = acc * s_val
                return y * (1.0 / (1.0 + jnp.exp(-jnp.clip(y, -60.0, 60.0))))

            out_t[:h, :] = silu(ra_t[...] + pa_t[:h, :])
            out_t[h:, :] = silu(rb_t[...] + pa_t[h:, :])

        pipe = pltpu.emit_pipeline(
            merge_tile,
            grid=(n // tile,),
            in_specs=[
                pl.BlockSpec((h, tile), lambda j: (0, j)),
                pl.BlockSpec((h, tile), lambda j: (0, j)),
                pl.BlockSpec((m, tile), lambda j: (d, j)),
            ],
            out_specs=[pl.BlockSpec((m, tile), lambda j: (d, j))],
        )
        pipe(ra_hbm, rb_hbm, p_hbm, out_ref)

        for s in range(N_DEV - 1):
            ia = lax.rem(d + N_DEV - s, N_DEV)
            ib = lax.rem(d + s, N_DEV)
            sl_a = (pl.ds(ia * m, h), slice(None))
            sl_b = (pl.ds(ib * m + h, h), slice(None))
            rdma_a = pltpu.make_async_remote_copy(
                src_ref=out_ref.at[sl_a], dst_ref=out_ref.at[sl_a],
                send_sem=sa.at[s], recv_sem=ra.at[s],
                device_id=(right,), device_id_type=pl.DeviceIdType.MESH,
            )
            rdma_b = pltpu.make_async_remote_copy(
                src_ref=out_ref.at[sl_b], dst_ref=out_ref.at[sl_b],
                send_sem=sb.at[s], recv_sem=rb.at[s],
                device_id=(left,), device_id_type=pl.DeviceIdType.MESH,
            )
            rdma_a.start()
            rdma_b.start()
            rdma_a.wait()
            rdma_b.wait()

    return pl.pallas_call(
        body,
        out_shape=jax.ShapeDtypeStruct((N_DEV * m, n), jnp.float32),
        in_specs=[
            pl.BlockSpec(memory_space=pltpu.MemorySpace.SMEM),
            pl.BlockSpec(memory_space=pl.ANY),
            pl.BlockSpec(memory_space=pl.ANY),
            pl.BlockSpec(memory_space=pl.ANY),
        ],
        out_specs=pl.BlockSpec(memory_space=pl.ANY),
        scratch_shapes=[pltpu.SemaphoreType.DMA((N_DEV - 1,))] * 4,
    )(scale, recv_a, recv_b, partial)


def kernel(x, w_mat, scale_x, scale_w):
    d = lax.axis_index("i")

    partial = jnp.dot(
        x.astype(jnp.bfloat16),
        w_mat.astype(jnp.bfloat16),
        preferred_element_type=jnp.float32,
    )

    m_tot, n = partial.shape
    m = m_tot // N_DEV
    h = m // 2

    def upper(i):
        return lax.dynamic_slice_in_dim(
            partial, lax.rem(i, N_DEV) * m, h, axis=0
        )

    def lower(i):
        return lax.dynamic_slice_in_dim(
            partial, lax.rem(i, N_DEV) * m + h, h, axis=0
        )

    cur_a = upper(d + (N_DEV - 1))
    cur_b = lower(d + 1)
    for s in range(N_DEV - 2):
        ra, rb = _ring_hop2(cur_a, cur_b)
        cur_a = ra + upper(d + (2 * N_DEV - 2 - s))
        cur_b = rb + lower(d + 2 + s)
    ra, rb = _ring_hop2(cur_a, cur_b)

    scale = (scale_x * scale_w).astype(jnp.float32)
    return _merge_all_gather(scale, ra, rb, partial, m, n)


# device time: 1191414 ns/iter; 1.1714x vs baseline; 1.1714x over previous
import jax
import jax.numpy as jnp
from jax import lax
from jax.experimental import pallas as pl
from jax.experimental.pallas import tpu as pltpu

N_DEV = 4
TILE = 1024


def _mono(scale, x, w):
    m_tot, _ = x.shape
    n = w.shape[1]
    m = m_tot // N_DEV
    h = m // 2
    nt = n // TILE

    def body(scale_ref, x_ref, w_ref,
             out_ref, sta, stb, rah, rbh,
             ca, cb, tmpa, tmpb, moa, mob,
             sem_sa, sem_ra, sem_sb, sem_rb,
             tl_a, tl_b, mo_a, mo_b,
             ag_sa, ag_ra, ag_sb, ag_rb):
        d = lax.axis_index("i")
        right = lax.rem(d + 1, N_DEV)
        left = lax.rem(d + N_DEV - 1, N_DEV)
        s_val = scale_ref[0]

        def dot_tile(c, lo, j):
            start = lax.rem(c, N_DEV) * m + (h if lo else 0)
            xa = x_ref[pl.ds(start, h), :].astype(jnp.bfloat16)
            wt = w_ref[:, j * TILE:(j + 1) * TILE].astype(jnp.bfloat16)
            return jnp.dot(xa, wt, preferred_element_type=jnp.float32)

        def rdma(src, dst, ssem, rsem, dev):
            return pltpu.make_async_remote_copy(
                src_ref=src, dst_ref=dst, send_sem=ssem, recv_sem=rsem,
                device_id=(dev,), device_id_type=pl.DeviceIdType.MESH)

        rds_a = [[None] * nt for _ in range(3)]
        rds_b = [[None] * nt for _ in range(3)]

        for j in range(nt):
            ca[j] = dot_tile(d + (N_DEV - 1), False, j)
            r = rdma(ca.at[j], sta.at[0, j],
                     sem_sa.at[0, j], sem_ra.at[0, j], right)
            r.start()
            rds_a[0][j] = r
            cb[j] = dot_tile(d + 1, True, j)
            r = rdma(cb.at[j], stb.at[0, j],
                     sem_sb.at[0, j], sem_rb.at[0, j], left)
            r.start()
            rds_b[0][j] = r

        for s in (1, 2):
            ca_chunk = d + (2 * N_DEV - 1 - s)
            cb_chunk = d + 1 + s
            for j in range(nt):
                rds_a[s - 1][j].wait()
                cpa = pltpu.make_async_copy(sta.at[s - 1, j], tmpa, tl_a)
                cpa.start()
                rds_b[s - 1][j].wait()
                cpb = pltpu.make_async_copy(stb.at[s - 1, j], tmpb, tl_b)
                cpb.start()
                cpa.wait()
                ca[j] = dot_tile(ca_chunk, False, j) + tmpa[...]
                dsta = sta.at[s, j] if s < 2 else rah.at[j]
                r = rdma(ca.at[j], dsta,
                         sem_sa.at[s, j], sem_ra.at[s, j], right)
                r.start()
                rds_a[s][j] = r
                cpb.wait()
                cb[j] = dot_tile(cb_chunk, True, j) + tmpb[...]
                dstb = stb.at[s, j] if s < 2 else rbh.at[j]
                r = rdma(cb.at[j], dstb,
                         sem_sb.at[s, j], sem_rb.at[s, j], left)
                r.start()
                rds_b[s][j] = r

        def silu(acc):
            y = acc * s_val
            return y * (1.0 / (1.0 + jnp.exp(-jnp.clip(y, -60.0, 60.0))))

        mo_cps = []
        for j in range(nt):
            rds_a[2][j].wait()
            cpa = pltpu.make_async_copy(rah.at[j], tmpa, tl_a)
            cpa.start()
            rds_b[2][j].wait()
            cpb = pltpu.make_async_copy(rbh.at[j], tmpb, tl_b)
            cpb.start()
            slot = j % 2
            if j >= 2:
                mo_cps[2 * (j - 2)].wait()
                mo_cps[2 * (j - 2) + 1].wait()
            cpa.wait()
            moa[slot] = silu(dot_tile(d, False, j) + tmpa[...])
            ka = pltpu.make_async_copy(
                moa.at[slot],
                out_ref.at[pl.ds(d * m, h), pl.ds(j * TILE, TILE)],
                mo_a.at[slot])
            ka.start()
            cpb.wait()
            mob[slot] = silu(dot_tile(d, True, j) + tmpb[...])
            kb = pltpu.make_async_copy(
                mob.at[slot],
                out_ref.at[pl.ds(d * m + h, h), pl.ds(j * TILE, TILE)],
                mo_b.at[slot])
            kb.start()
            mo_cps += [ka, kb]
        for cp in mo_cps[2 * (nt - 2):]:
            cp.wait()

        for s in range(N_DEV - 1):
            ia = lax.rem(d + N_DEV - s, N_DEV)
            ib = lax.rem(d + s, N_DEV)
            sl_a = (pl.ds(ia * m, h), slice(None))
            sl_b = (pl.ds(ib * m + h, h), slice(None))
            rdma_a = rdma(out_ref.at[sl_a], out_ref.at[sl_a],
                          ag_sa.at[s], ag_ra.at[s], right)
            rdma_b = rdma(out_ref.at[sl_b], out_ref.at[sl_b],
                          ag_sb.at[s], ag_rb.at[s], left)
            rdma_a.start()
            rdma_b.start()
            rdma_a.wait()
            rdma_b.wait()

    out, *_ = pl.pallas_call(
        body,
        out_shape=(
            jax.ShapeDtypeStruct((m_tot, n), jnp.float32),
            jax.ShapeDtypeStruct((2, nt, h, TILE), jnp.float32),
            jax.ShapeDtypeStruct((2, nt, h, TILE), jnp.float32),
            jax.ShapeDtypeStruct((nt, h, TILE), jnp.float32),
            jax.ShapeDtypeStruct((nt, h, TILE), jnp.float32),
        ),
        in_specs=[
            pl.BlockSpec(memory_space=pltpu.MemorySpace.SMEM),
            pl.BlockSpec(memory_space=pltpu.MemorySpace.VMEM),
            pl.BlockSpec(memory_space=pltpu.MemorySpace.VMEM),
        ],
        out_specs=tuple(pl.BlockSpec(memory_space=pl.ANY) for _ in range(5)),
        scratch_shapes=[
            pltpu.MemorySpace.VMEM((nt, h, TILE), jnp.float32),
            pltpu.MemorySpace.VMEM((nt, h, TILE), jnp.float32),
            pltpu.MemorySpace.VMEM((h, TILE), jnp.float32),
            pltpu.MemorySpace.VMEM((h, TILE), jnp.float32),
            pltpu.MemorySpace.VMEM((2, h, TILE), jnp.float32),
            pltpu.MemorySpace.VMEM((2, h, TILE), jnp.float32),
            pltpu.SemaphoreType.DMA((3, nt)),
            pltpu.SemaphoreType.DMA((3, nt)),
            pltpu.SemaphoreType.DMA((3, nt)),
            pltpu.SemaphoreType.DMA((3, nt)),
            pltpu.SemaphoreType.DMA,
            pltpu.SemaphoreType.DMA,
            pltpu.SemaphoreType.DMA((2,)),
            pltpu.SemaphoreType.DMA((2,)),
            pltpu.SemaphoreType.DMA((N_DEV - 1,)),
            pltpu.SemaphoreType.DMA((N_DEV - 1,)),
            pltpu.SemaphoreType.DMA((N_DEV - 1,)),
            pltpu.SemaphoreType.DMA((N_DEV - 1,)),
        ],
        compiler_params=pltpu.CompilerParams(
            vmem_limit_bytes=100 * 1024 * 1024,
        ),
    )(scale, x, w)
    return out


def kernel(x, w_mat, scale_x, scale_w):
    scale = (scale_x * scale_w).astype(jnp.float32)
    return _mono(scale, x, w_mat)


# device time: 1182018 ns/iter; 1.1807x vs baseline; 1.0079x over previous
import jax
import jax.numpy as jnp
from jax import lax
from jax.experimental import pallas as pl
from jax.experimental.pallas import tpu as pltpu

N_DEV = 4
TILE = 1024
SR = 128


def _mono(scale, x, w):
    m_tot, _ = x.shape
    n = w.shape[1]
    m = m_tot // N_DEV
    h = m // 2
    nt = n // TILE
    sn = h // SR

    def body(scale_ref, x_ref, w_ref,
             out_ref, sta, stb, rah, rbh,
             ca, cb, moa, mob,
             sem_sa, sem_ra, sem_sb, sem_rb,
             tl_a, tl_b, mo_a, mo_b,
             ag_sa, ag_ra, ag_sb, ag_rb):
        d = lax.axis_index("i")
        right = lax.rem(d + 1, N_DEV)
        left = lax.rem(d + N_DEV - 1, N_DEV)
        s_val = scale_ref[0]

        def gemm_strip(buf, idx, c, lo, r, accumulate):
            start = lax.rem(c, N_DEV) * m + (h if lo else 0) + r * SR
            xa = x_ref[pl.ds(start, SR), :].astype(jnp.bfloat16)
            for j in range(nt):
                js = slice(j * TILE, (j + 1) * TILE)
                wt = w_ref[:, js].astype(jnp.bfloat16)
                dv = jnp.dot(xa, wt, preferred_element_type=jnp.float32)
                if accumulate:
                    buf[idx, :, js] = buf[idx, :, js] + dv
                else:
                    buf[idx, :, js] = dv

        def merge_strip(buf, lo, r):
            start = d * m + (h if lo else 0) + r * SR
            xa = x_ref[pl.ds(start, SR), :].astype(jnp.bfloat16)
            for j in range(nt):
                js = slice(j * TILE, (j + 1) * TILE)
                wt = w_ref[:, js].astype(jnp.bfloat16)
                dv = jnp.dot(xa, wt, preferred_element_type=jnp.float32)
                y = (buf[:, js] + dv) * s_val
                buf[:, js] = y * (1.0 / (1.0 + jnp.exp(-jnp.clip(y, -60.0, 60.0))))

        def rdma(src, dst, ssem, rsem, dev):
            return pltpu.make_async_remote_copy(
                src_ref=src, dst_ref=dst, send_sem=ssem, recv_sem=rsem,
                device_id=(dev,), device_id_type=pl.DeviceIdType.MESH)

        rds_a = [[None] * sn for _ in range(3)]
        rds_b = [[None] * sn for _ in range(3)]

        for r in range(sn):
            gemm_strip(ca, r, d + (N_DEV - 1), False, r, False)
            rd = rdma(ca.at[r], sta.at[0, r],
                      sem_sa.at[0, r], sem_ra.at[0, r], right)
            rd.start()
            rds_a[0][r] = rd
            gemm_strip(cb, r, d + 1, True, r, False)
            rd = rdma(cb.at[r], stb.at[0, r],
                      sem_sb.at[0, r], sem_rb.at[0, r], left)
            rd.start()
            rds_b[0][r] = rd

        for s in (1, 2):
            c_a = d + (2 * N_DEV - 1 - s)
            c_b = d + 1 + s
            for r in range(sn):
                rds_a[s - 1][r].wait()
                cpa = pltpu.make_async_copy(sta.at[s - 1, r], ca.at[r], tl_a)
                cpa.start()
                rds_b[s - 1][r].wait()
                cpb = pltpu.make_async_copy(stb.at[s - 1, r], cb.at[r], tl_b)
                cpb.start()
                cpa.wait()
                gemm_strip(ca, r, c_a, False, r, True)
                dsta = sta.at[s, r] if s < 2 else rah.at[r]
                rd = rdma(ca.at[r], dsta,
                          sem_sa.at[s, r], sem_ra.at[s, r], right)
                rd.start()
                rds_a[s][r] = rd
                cpb.wait()
                gemm_strip(cb, r, c_b, True, r, True)
                dstb = stb.at[s, r] if s < 2 else rbh.at[r]
                rd = rdma(cb.at[r], dstb,
                          sem_sb.at[s, r], sem_rb.at[s, r], left)
                rd.start()
                rds_b[s][r] = rd

        ag_a = [[None] * sn for _ in range(3)]
        ag_b = [[None] * sn for _ in range(3)]

        def ag_hop(s2, r):
            ia = lax.rem(d + N_DEV - s2, N_DEV)
            ib = lax.rem(d + s2, N_DEV)
            sl_a = (pl.ds(ia * m + r * SR, SR), slice(None))
            sl_b = (pl.ds(ib * m + h + r * SR, SR), slice(None))
            rd = rdma(out_ref.at[sl_a], out_ref.at[sl_a],
                      ag_sa.at[s2, r], ag_ra.at[s2, r], right)
            rd.start()
            ag_a[s2][r] = rd
            rd = rdma(out_ref.at[sl_b], out_ref.at[sl_b],
                      ag_sb.at[s2, r], ag_rb.at[s2, r], left)
            rd.start()
            ag_b[s2][r] = rd

        mo_cps_a, mo_cps_b = [], []
        for r in range(sn):
            rds_a[2][r].wait()
            if r >= 1:
                mo_cps_a[r - 1].wait()
                mo_cps_b[r - 1].wait()
                ag_hop(0, r - 1)
            cpa = pltpu.make_async_copy(rah.at[r], moa, tl_a)
            cpa.start()
            rds_b[2][r].wait()
            cpb = pltpu.make_async_copy(rbh.at[r], mob, tl_b)
            cpb.start()
            cpa.wait()
            merge_strip(moa, False, r)
            ka = pltpu.make_async_copy(
                moa, out_ref.at[pl.ds(d * m + r * SR, SR), :], mo_a.at[r])
            ka.start()
            mo_cps_a.append(ka)
            cpb.wait()
            merge_strip(mob, True, r)
            kb = pltpu.make_async_copy(
                mob, out_ref.at[pl.ds(d * m + h + r * SR, SR), :], mo_b.at[r])
            kb.start()
            mo_cps_b.append(kb)
        mo_cps_a[sn - 1].wait()
        mo_cps_b[sn - 1].wait()
        ag_hop(0, sn - 1)

        for s2 in (1, 2):
            for r in range(sn):
                ag_a[s2 - 1][r].wait()
                ag_b[s2 - 1][r].wait()
                ag_hop(s2, r)
        for r in range(sn):
            ag_a[2][r].wait()
            ag_b[2][r].wait()

    out, *_ = pl.pallas_call(
        body,
        out_shape=(
            jax.ShapeDtypeStruct((m_tot, n), jnp.float32),
            jax.ShapeDtypeStruct((2, sn, SR, n), jnp.float32),
            jax.ShapeDtypeStruct((2, sn, SR, n), jnp.float32),
            jax.ShapeDtypeStruct((sn, SR, n), jnp.float32),
            jax.ShapeDtypeStruct((sn, SR, n), jnp.float32),
        ),
        in_specs=[
            pl.BlockSpec(memory_space=pltpu.MemorySpace.SMEM),
            pl.BlockSpec(memory_space=pltpu.MemorySpace.VMEM),
            pl.BlockSpec(memory_space=pltpu.MemorySpace.VMEM),
        ],
        out_specs=tuple(pl.BlockSpec(memory_space=pl.ANY) for _ in range(5)),
        scratch_shapes=[
            pltpu.MemorySpace.VMEM((sn, SR, n), jnp.float32),
            pltpu.MemorySpace.VMEM((sn, SR, n), jnp.float32),
            pltpu.MemorySpace.VMEM((SR, n), jnp.float32),
            pltpu.MemorySpace.VMEM((SR, n), jnp.float32),
            pltpu.SemaphoreType.DMA((3, sn)),
            pltpu.SemaphoreType.DMA((3, sn)),
            pltpu.SemaphoreType.DMA((3, sn)),
            pltpu.SemaphoreType.DMA((3, sn)),
            pltpu.SemaphoreType.DMA,
            pltpu.SemaphoreType.DMA,
            pltpu.SemaphoreType.DMA((sn,)),
            pltpu.SemaphoreType.DMA((sn,)),
            pltpu.SemaphoreType.DMA((3, sn)),
            pltpu.SemaphoreType.DMA((3, sn)),
            pltpu.SemaphoreType.DMA((3, sn)),
            pltpu.SemaphoreType.DMA((3, sn)),
        ],
        compiler_params=pltpu.CompilerParams(
            vmem_limit_bytes=100 * 1024 * 1024,
        ),
    )(scale, x, w)
    return out


def kernel(x, w_mat, scale_x, scale_w):
    scale = (scale_x * scale_w).astype(jnp.float32)
    return _mono(scale, x, w_mat)
